# baseline (speedup 1.0000x reference)
import dataclasses
import functools

import jax
import jax.numpy as jnp
from jax import lax
from jax.experimental import pallas as pl
from jax.experimental.pallas import tpu as pltpu
from jax.experimental.pallas import tpu_sc as plsc

NC = 2
NS = 16
NW = NC * NS
CH = 40


def _make_spmm(n_nodes, n_edges, d):
    epw = n_edges // NW
    ibc = 25
    ib = ibc * CH
    nib = epw // ib
    assert epw * NW == n_edges and nib * ib == epw
    nrch = n_nodes // CH
    assert nrch * CH == n_nodes
    mesh = plsc.VectorSubcoreMesh(core_axis_name="c", subcore_axis_name="s")
    cp = pltpu.CompilerParams()
    if "needs_layout_passes" in pltpu.CompilerParams.__dataclass_fields__:
        cp = dataclasses.replace(cp, needs_layout_passes=False)
    if d % 128 != 0:
        cp = dataclasses.replace(cp, use_tc_tiling_on_sc=False)

    nch = epw // CH
    assert nch % 2 == 0
    scr = (2 * ibc, CH)
    cwp = (CH + 15) // 16 * 16

    @functools.partial(
        pl.kernel,
        out_type=jax.ShapeDtypeStruct((NC, n_nodes, d), jnp.float32),
        mesh=mesh,
        compiler_params=cp,
        scratch_types=[
            pltpu.VMEM(scr, jnp.int32),
            pltpu.VMEM(scr, jnp.int32),
            pltpu.VMEM((2 * ibc, cwp), jnp.float32),
            pltpu.VMEM((2, CH, d), jnp.float32),
            pltpu.VMEM((2, CH, d), jnp.float32),
            pltpu.VMEM_SHARED((n_nodes, d), jnp.float32),
            pltpu.SemaphoreType.DMA,
            pltpu.SemaphoreType.DMA,
            pltpu.SemaphoreType.DMA,
            pltpu.SemaphoreType.DMA,
            pltpu.SemaphoreType.DMA,
        ],
    )
    def spmm(h_hbm, src_hbm, dst_hbm, w_hbm, out_hbm,
             src_v, dst_v, w_v, rows_v, scaled_v, acc,
             isem, gsem0, gsem1, ssem0, ssem1):
        c = lax.axis_index("c")
        s = lax.axis_index("s")
        wid = s * NC + c
        gsems = (gsem0, gsem1)
        ssems = (ssem0, ssem1)

        def stage(sl, b):
            blk = wid * nib + b
            row = pl.ds(sl * ibc, ibc)
            pltpu.async_copy(src_hbm.at[blk], src_v.at[row], isem)
            pltpu.async_copy(dst_hbm.at[blk], dst_v.at[row], isem)
            pltpu.async_copy(w_hbm.at[blk], w_v.at[row], isem)

        def wait_stage(sl):
            row = pl.ds(sl * ibc, ibc)
            pltpu.make_async_copy(src_hbm.at[0], src_v.at[row], isem).wait()
            pltpu.make_async_copy(dst_hbm.at[0], dst_v.at[row], isem).wait()
            pltpu.make_async_copy(w_hbm.at[0], w_v.at[row], isem).wait()

        def issue_gather(cc, p):
            srow = (cc // ibc) % 2 * ibc + cc % ibc
            pltpu.async_copy(h_hbm.at[src_v.at[srow]], rows_v.at[p], gsems[p])

        def wait_gather(p):
            pltpu.make_async_copy(out_hbm.at[0, pl.ds(0, CH)],
                                  rows_v.at[p], gsems[p]).wait()

        def issue_scatter(cc, p):
            drow = (cc // ibc) % 2 * ibc + cc % ibc
            pltpu.async_copy(scaled_v.at[p], acc.at[dst_v.at[drow]],
                             ssems[p], add=True)

        def wait_scatter(p):
            pltpu.make_async_copy(scaled_v.at[p], acc.at[pl.ds(0, CH)],
                                  ssems[p]).wait()

        stage(0, 0)
        wait_stage(0)
        issue_gather(0, 0)
        issue_gather(1, 1)

        zv = jnp.zeros((16,), jnp.float32)

        @pl.loop(0, CH)
        def _(r):
            for k in range(d // 16):
                scaled_v[0, r, pl.ds(k * 16, 16)] = zv

        @pl.loop(s, nrch, step=NS)
        def _(q):
            pltpu.sync_copy(scaled_v.at[0], acc.at[pl.ds(q * CH, CH)])
        plsc.subcore_barrier()

        dnums = lax.GatherDimensionNumbers(
            offset_dims=(), collapsed_slice_dims=(0,), start_index_map=(0,))

        def splat(vec, i):
            idx = jnp.full((16, 1), i, jnp.int32)
            return lax.gather(vec, idx, dnums, slice_sizes=(1,),
                              mode=lax.GatherScatterMode.PROMISE_IN_BOUNDS)

        def scale(cc, p):
            wrow = (cc // ibc) % 2 * ibc + cc % ibc
            for g in range(cwp // 16):
                wv = w_v[wrow, pl.ds(g * 16, 16)]
                for i in range(min(16, CH - g * 16)):
                    e = g * 16 + i
                    wb = splat(wv, i)
                    for k in range(d // 16):
                        slc = pl.ds(k * 16, 16)
                        scaled_v[p, e, slc] = rows_v[p, e, slc] * wb

        def step(cc, p, tail):
            wait_gather(p)

            @pl.when((cc % ibc == 2) & (cc < (nib - 1) * ibc))
            def _():
                stage((cc // ibc + 1) % 2, cc // ibc + 1)

            @pl.when(cc >= 2)
            def _():
                wait_scatter(p)

            scale(cc, p)
            if not tail:
                @pl.when((cc + 2) % ibc == 0)
                def _():
                    wait_stage(((cc + 2) // ibc) % 2)
                issue_gather(cc + 2, p)
            issue_scatter(cc, p)

        @pl.loop(0, nch - 2, step=2)
        def _(j):
            step(j, 0, False)
            step(j + 1, 1, False)

        step(nch - 2, 0, True)
        step(nch - 1, 1, True)
        wait_scatter(0)
        wait_scatter(1)
        plsc.subcore_barrier()

        @pl.loop(s, nrch, step=NS)
        def _(q):
            pltpu.sync_copy(acc.at[pl.ds(q * CH, CH)],
                            out_hbm.at[c, pl.ds(q * CH, CH)])

    return spmm


def _matmul_tc(x, w):
    def body(x_ref, w_ref, o_ref):
        o_ref[...] = jnp.dot(x_ref[...], w_ref[...],
                             preferred_element_type=jnp.float32)

    return pl.pallas_call(
        body,
        out_shape=jax.ShapeDtypeStruct((x.shape[0], w.shape[1]), jnp.float32),
    )(x, w)


def _mid_tc(p, w1):
    n = p.shape[1]

    def body(p_ref, w1_ref, o_ref):
        h = jnp.maximum(p_ref[0] + p_ref[1], 0.0)
        o_ref[...] = jnp.dot(h, w1_ref[...],
                             preferred_element_type=jnp.float32)

    return pl.pallas_call(
        body,
        out_shape=jax.ShapeDtypeStruct((n, w1.shape[1]), jnp.float32),
    )(p, w1)


def _loss_tc(q, label, mask2d):
    def body(q_ref, y_ref, m_ref, loss_ref, acc_ref):
        logits = q_ref[0] + q_ref[1]
        y = y_ref[...]
        m = m_ref[...]
        k = logits.shape[1]
        rowmax = jnp.max(logits, axis=1, keepdims=True)
        sh = logits - rowmax
        lse = jnp.log(jnp.sum(jnp.exp(sh), axis=1, keepdims=True))
        ce = -jnp.sum(y * (sh - lse), axis=1, keepdims=True)
        ii = lax.broadcasted_iota(jnp.int32, logits.shape, 1)
        am_l = jnp.min(jnp.where(logits >= rowmax, ii, k), axis=1,
                       keepdims=True)
        ymax = jnp.max(y, axis=1, keepdims=True)
        am_y = jnp.min(jnp.where(y >= ymax, ii, k), axis=1, keepdims=True)
        correct = (am_l == am_y).astype(jnp.float32)
        msum = jnp.sum(m)
        loss_ref[...] = (jnp.sum(ce * m) / msum).reshape(1, 1)
        acc_ref[...] = (jnp.sum(correct * m) / msum).reshape(1, 1)

    return pl.pallas_call(
        body,
        out_shape=(jax.ShapeDtypeStruct((1, 1), jnp.float32),
                   jax.ShapeDtypeStruct((1, 1), jnp.float32)),
    )(q, label, mask2d)


def kernel(x, label, mask, edge_index, edge_weight, W0, W1):
    n, d_in = x.shape
    e = edge_index.shape[1]
    src = edge_index[0].astype(jnp.int32)
    dst = edge_index[1].astype(jnp.int32)
    epw = e // NW
    ibc = 25
    nblk = e // (ibc * CH)
    srcr = src.reshape(nblk, ibc, CH)
    dstr = dst.reshape(nblk, ibc, CH)
    cwp = (CH + 15) // 16 * 16
    wr = jnp.pad(edge_weight.reshape(nblk, ibc, CH),
                 ((0, 0), (0, 0), (0, cwp - CH)))

    xw = _matmul_tc(x, W0)
    p = _make_spmm(n, e, W0.shape[1])(xw, srcr, dstr, wr)
    h1 = _mid_tc(p, W1)
    q = _make_spmm(n, e, W1.shape[1])(h1, srcr, dstr, wr)
    loss2, acc2 = _loss_tc(q, label, mask.reshape(n, 1))
    return (loss2[0, 0], acc2[0, 0])


# speedup vs baseline: 1.1068x; 1.1068x over previous
import dataclasses
import functools

import jax
import jax.numpy as jnp
from jax import lax
from jax.experimental import pallas as pl
from jax.experimental.pallas import tpu as pltpu
from jax.experimental.pallas import tpu_sc as plsc

NC = 2
NS = 16
NW = NC * NS
CH = 80


def _make_spmm(n_nodes, n_edges, d):
    epw = n_edges // NW
    ibc = 5
    ib = ibc * CH
    nib = epw // ib
    assert epw * NW == n_edges and nib * ib == epw
    nrch = n_nodes // CH
    assert nrch * CH == n_nodes
    mesh = plsc.VectorSubcoreMesh(core_axis_name="c", subcore_axis_name="s")
    cp = pltpu.CompilerParams()
    if "needs_layout_passes" in pltpu.CompilerParams.__dataclass_fields__:
        cp = dataclasses.replace(cp, needs_layout_passes=False)
    if d % 128 != 0:
        cp = dataclasses.replace(cp, use_tc_tiling_on_sc=False)

    nch = epw // CH
    assert (nch - 2) % 3 == 0
    scr = (3 * ibc, CH)
    cwp = (CH + 15) // 16 * 16

    @functools.partial(
        pl.kernel,
        out_type=jax.ShapeDtypeStruct((NC, n_nodes, d), jnp.float32),
        mesh=mesh,
        compiler_params=cp,
        scratch_types=[
            pltpu.VMEM(scr, jnp.int32),
            pltpu.VMEM(scr, jnp.int32),
            pltpu.VMEM((3 * ibc, cwp), jnp.float32),
            pltpu.VMEM((3, CH, d), jnp.float32),
            pltpu.VMEM_SHARED((n_nodes, d), jnp.float32),
            pltpu.SemaphoreType.DMA,
            pltpu.SemaphoreType.DMA,
            pltpu.SemaphoreType.DMA,
            pltpu.SemaphoreType.DMA,
            pltpu.SemaphoreType.DMA,
            pltpu.SemaphoreType.DMA,
            pltpu.SemaphoreType.DMA,
        ],
    )
    def spmm(h_hbm, src_hbm, dst_hbm, w_hbm, out_hbm,
             src_v, dst_v, w_v, rows_v, acc,
             isem, gsem0, gsem1, gsem2, ssem0, ssem1, ssem2):
        c = lax.axis_index("c")
        s = lax.axis_index("s")
        wid = s * NC + c
        gsems = (gsem0, gsem1, gsem2)
        ssems = (ssem0, ssem1, ssem2)

        def stage(sl, b):
            blk = wid * nib + b
            row = pl.ds(sl * ibc, ibc)
            pltpu.async_copy(src_hbm.at[blk], src_v.at[row], isem)
            pltpu.async_copy(dst_hbm.at[blk], dst_v.at[row], isem)
            pltpu.async_copy(w_hbm.at[blk], w_v.at[row], isem)

        def wait_stage(sl):
            row = pl.ds(sl * ibc, ibc)
            pltpu.make_async_copy(src_hbm.at[0], src_v.at[row], isem).wait()
            pltpu.make_async_copy(dst_hbm.at[0], dst_v.at[row], isem).wait()
            pltpu.make_async_copy(w_hbm.at[0], w_v.at[row], isem).wait()

        def issue_gather(cc, p):
            srow = (cc // ibc) % 3 * ibc + cc % ibc
            pltpu.async_copy(h_hbm.at[src_v.at[srow]], rows_v.at[p], gsems[p])

        def wait_gather(p):
            pltpu.make_async_copy(out_hbm.at[0, pl.ds(0, CH)],
                                  rows_v.at[p], gsems[p]).wait()

        def issue_scatter(cc, p):
            drow = (cc // ibc) % 3 * ibc + cc % ibc
            pltpu.async_copy(rows_v.at[p], acc.at[dst_v.at[drow]],
                             ssems[p], add=True)

        def wait_scatter(p):
            pltpu.make_async_copy(rows_v.at[p], acc.at[pl.ds(0, CH)],
                                  ssems[p]).wait()

        stage(0, 0)
        stage(1, 1)
        wait_stage(0)
        issue_gather(0, 0)
        issue_gather(1, 1)

        zv = jnp.zeros((16,), jnp.float32)

        @pl.loop(0, CH)
        def _(r):
            for k in range(d // 16):
                rows_v[2, r, pl.ds(k * 16, 16)] = zv

        @pl.loop(s, nrch, step=NS)
        def _(q):
            pltpu.sync_copy(rows_v.at[2], acc.at[pl.ds(q * CH, CH)])
        plsc.subcore_barrier()

        dnums = lax.GatherDimensionNumbers(
            offset_dims=(), collapsed_slice_dims=(0,), start_index_map=(0,))

        def splat(vec, i):
            idx = jnp.full((16, 1), i, jnp.int32)
            return lax.gather(vec, idx, dnums, slice_sizes=(1,),
                              mode=lax.GatherScatterMode.PROMISE_IN_BOUNDS)

        def scale(cc, p):
            wrow = (cc // ibc) % 3 * ibc + cc % ibc
            for g in range(cwp // 16):
                wv = w_v[wrow, pl.ds(g * 16, 16)]
                for i in range(min(16, CH - g * 16)):
                    e = g * 16 + i
                    wb = splat(wv, i)
                    for k in range(d // 16):
                        slc = pl.ds(k * 16, 16)
                        rows_v[p, e, slc] = rows_v[p, e, slc] * wb

        def step(cc, p, tail):
            wait_gather(p)

            @pl.when((cc % ibc == 2) & (cc < (nib - 2) * ibc))
            def _():
                stage((cc // ibc + 2) % 3, cc // ibc + 2)

            scale(cc, p)
            q = (p + 2) % 3

            @pl.when(cc >= 1)
            def _():
                wait_scatter(q)

            if not tail:
                @pl.when((cc + 2) % ibc == 0)
                def _():
                    wait_stage(((cc + 2) // ibc) % 3)
                issue_gather(cc + 2, q)
            issue_scatter(cc, p)

        @pl.loop(0, nch - 2, step=3)
        def _(j):
            step(j, 0, False)
            step(j + 1, 1, False)
            step(j + 2, 2, False)

        step(nch - 2, 0, True)
        step(nch - 1, 1, True)
        wait_scatter(1)
        plsc.subcore_barrier()

        @pl.loop(s, nrch, step=NS)
        def _(q):
            pltpu.sync_copy(acc.at[pl.ds(q * CH, CH)],
                            out_hbm.at[c, pl.ds(q * CH, CH)])

    return spmm


def _matmul_tc(x, w):
    def body(x_ref, w_ref, o_ref):
        o_ref[...] = jnp.dot(x_ref[...], w_ref[...],
                             preferred_element_type=jnp.float32)

    return pl.pallas_call(
        body,
        out_shape=jax.ShapeDtypeStruct((x.shape[0], w.shape[1]), jnp.float32),
    )(x, w)


def _mid_tc(p, w1):
    n = p.shape[1]

    def body(p_ref, w1_ref, o_ref):
        h = jnp.maximum(p_ref[0] + p_ref[1], 0.0)
        o_ref[...] = jnp.dot(h, w1_ref[...],
                             preferred_element_type=jnp.float32)

    return pl.pallas_call(
        body,
        out_shape=jax.ShapeDtypeStruct((n, w1.shape[1]), jnp.float32),
    )(p, w1)


def _loss_tc(q, label, mask2d):
    def body(q_ref, y_ref, m_ref, loss_ref, acc_ref):
        logits = q_ref[0] + q_ref[1]
        y = y_ref[...]
        m = m_ref[...]
        k = logits.shape[1]
        rowmax = jnp.max(logits, axis=1, keepdims=True)
        sh = logits - rowmax
        lse = jnp.log(jnp.sum(jnp.exp(sh), axis=1, keepdims=True))
        ce = -jnp.sum(y * (sh - lse), axis=1, keepdims=True)
        ii = lax.broadcasted_iota(jnp.int32, logits.shape, 1)
        am_l = jnp.min(jnp.where(logits >= rowmax, ii, k), axis=1,
                       keepdims=True)
        ymax = jnp.max(y, axis=1, keepdims=True)
        am_y = jnp.min(jnp.where(y >= ymax, ii, k), axis=1, keepdims=True)
        correct = (am_l == am_y).astype(jnp.float32)
        msum = jnp.sum(m)
        loss_ref[...] = (jnp.sum(ce * m) / msum).reshape(1, 1)
        acc_ref[...] = (jnp.sum(correct * m) / msum).reshape(1, 1)

    return pl.pallas_call(
        body,
        out_shape=(jax.ShapeDtypeStruct((1, 1), jnp.float32),
                   jax.ShapeDtypeStruct((1, 1), jnp.float32)),
    )(q, label, mask2d)


def kernel(x, label, mask, edge_index, edge_weight, W0, W1):
    n, d_in = x.shape
    e = edge_index.shape[1]
    src = edge_index[0].astype(jnp.int32)
    dst = edge_index[1].astype(jnp.int32)
    epw = e // NW
    ibc = 5
    nblk = e // (ibc * CH)
    srcr = src.reshape(nblk, ibc, CH)
    dstr = dst.reshape(nblk, ibc, CH)
    cwp = (CH + 15) // 16 * 16
    wr = jnp.pad(edge_weight.reshape(nblk, ibc, CH),
                 ((0, 0), (0, 0), (0, cwp - CH)))

    xw = _matmul_tc(x, W0)
    p = _make_spmm(n, e, W0.shape[1])(xw, srcr, dstr, wr)
    h1 = _mid_tc(p, W1)
    q = _make_spmm(n, e, W1.shape[1])(h1, srcr, dstr, wr)
    loss2, acc2 = _loss_tc(q, label, mask.reshape(n, 1))
    return (loss2[0, 0], acc2[0, 0])


# speedup vs baseline: 1.1343x; 1.0248x over previous
import dataclasses
import functools

import jax
import jax.numpy as jnp
from jax import lax
from jax.experimental import pallas as pl
from jax.experimental.pallas import tpu as pltpu
from jax.experimental.pallas import tpu_sc as plsc

NC = 2
NS = 16
NW = NC * NS
CH = 80


def _make_spmm(n_nodes, n_edges, d):
    epw = n_edges // NW
    ibc = 5
    ib = ibc * CH
    nib = epw // ib
    assert epw * NW == n_edges and nib * ib == epw
    nrch = n_nodes // CH
    assert nrch * CH == n_nodes
    mesh = plsc.VectorSubcoreMesh(core_axis_name="c", subcore_axis_name="s")
    cp = pltpu.CompilerParams()
    if "needs_layout_passes" in pltpu.CompilerParams.__dataclass_fields__:
        cp = dataclasses.replace(cp, needs_layout_passes=False)
    cp = dataclasses.replace(cp, use_tc_tiling_on_sc=False)

    nch = epw // CH
    assert (nch - 2) % 3 == 0
    scr = (3 * ibc, CH)
    cwp = (CH + 15) // 16 * 16

    @functools.partial(
        pl.kernel,
        out_type=jax.ShapeDtypeStruct((NC, n_nodes, d), jnp.float32),
        mesh=mesh,
        compiler_params=cp,
        scratch_types=[
            pltpu.VMEM(scr, jnp.int32),
            pltpu.VMEM(scr, jnp.int32),
            pltpu.VMEM((3 * ibc, cwp), jnp.float32),
            pltpu.VMEM((3, CH, d), jnp.float32),
            pltpu.VMEM_SHARED((n_nodes, d), jnp.float32),
            pltpu.SemaphoreType.DMA,
            pltpu.SemaphoreType.DMA,
            pltpu.SemaphoreType.DMA,
            pltpu.SemaphoreType.DMA,
            pltpu.SemaphoreType.DMA,
            pltpu.SemaphoreType.DMA,
            pltpu.SemaphoreType.DMA,
        ],
    )
    def spmm(h_hbm, src_hbm, dst_hbm, w_hbm, out_hbm,
             src_v, dst_v, w_v, rows_v, acc,
             isem, gsem0, gsem1, gsem2, ssem0, ssem1, ssem2):
        c = lax.axis_index("c")
        s = lax.axis_index("s")
        wid = s * NC + c
        gsems = (gsem0, gsem1, gsem2)
        ssems = (ssem0, ssem1, ssem2)

        def stage(sl, b):
            blk = wid * nib + b
            row = pl.ds(sl * ibc, ibc)
            pltpu.async_copy(src_hbm.at[blk], src_v.at[row], isem)
            pltpu.async_copy(dst_hbm.at[blk], dst_v.at[row], isem)
            pltpu.async_copy(w_hbm.at[blk], w_v.at[row], isem)

        def wait_stage(sl):
            row = pl.ds(sl * ibc, ibc)
            pltpu.make_async_copy(src_hbm.at[0], src_v.at[row], isem).wait()
            pltpu.make_async_copy(dst_hbm.at[0], dst_v.at[row], isem).wait()
            pltpu.make_async_copy(w_hbm.at[0], w_v.at[row], isem).wait()

        def issue_gather(cc, p):
            srow = (cc // ibc) % 3 * ibc + cc % ibc
            pltpu.async_copy(h_hbm.at[src_v.at[srow]], rows_v.at[p], gsems[p])

        def wait_gather(p):
            pltpu.make_async_copy(out_hbm.at[0, pl.ds(0, CH)],
                                  rows_v.at[p], gsems[p]).wait()

        def issue_scatter(cc, p):
            drow = (cc // ibc) % 3 * ibc + cc % ibc
            pltpu.async_copy(rows_v.at[p], acc.at[dst_v.at[drow]],
                             ssems[p], add=True)

        def wait_scatter(p):
            pltpu.make_async_copy(rows_v.at[p], acc.at[pl.ds(0, CH)],
                                  ssems[p]).wait()

        stage(0, 0)
        stage(1, 1)
        wait_stage(0)
        issue_gather(0, 0)
        issue_gather(1, 1)

        zv = jnp.zeros((16,), jnp.float32)

        @pl.loop(0, CH)
        def _(r):
            for k in range(d // 16):
                rows_v[2, r, pl.ds(k * 16, 16)] = zv

        @pl.loop(s, nrch, step=NS)
        def _(q):
            pltpu.sync_copy(rows_v.at[2], acc.at[pl.ds(q * CH, CH)])
        plsc.subcore_barrier()

        dnums = lax.GatherDimensionNumbers(
            offset_dims=(), collapsed_slice_dims=(0,), start_index_map=(0,))

        def splat(vec, i):
            idx = jnp.full((16, 1), i, jnp.int32)
            return lax.gather(vec, idx, dnums, slice_sizes=(1,),
                              mode=lax.GatherScatterMode.PROMISE_IN_BOUNDS)

        def scale(cc, p):
            wrow = (cc // ibc) % 3 * ibc + cc % ibc
            for g in range(cwp // 16):
                wv = w_v[wrow, pl.ds(g * 16, 16)]
                for i in range(min(16, CH - g * 16)):
                    e = g * 16 + i
                    wb = splat(wv, i)
                    for k in range(d // 16):
                        slc = pl.ds(k * 16, 16)
                        rows_v[p, e, slc] = rows_v[p, e, slc] * wb

        def step(cc, p, tail):
            wait_gather(p)

            @pl.when((cc % ibc == 2) & (cc < (nib - 2) * ibc))
            def _():
                stage((cc // ibc + 2) % 3, cc // ibc + 2)

            scale(cc, p)
            q = (p + 2) % 3

            @pl.when(cc >= 1)
            def _():
                wait_scatter(q)

            if not tail:
                @pl.when((cc + 2) % ibc == 0)
                def _():
                    wait_stage(((cc + 2) // ibc) % 3)
                issue_gather(cc + 2, q)
            issue_scatter(cc, p)

        @pl.loop(0, nch - 2, step=3)
        def _(j):
            step(j, 0, False)
            step(j + 1, 1, False)
            step(j + 2, 2, False)

        step(nch - 2, 0, True)
        step(nch - 1, 1, True)
        wait_scatter(1)
        plsc.subcore_barrier()

        @pl.loop(s, nrch, step=NS)
        def _(q):
            pltpu.sync_copy(acc.at[pl.ds(q * CH, CH)],
                            out_hbm.at[c, pl.ds(q * CH, CH)])

    return spmm


def _matmul_tc(x, w):
    def body(x_ref, w_ref, o_ref):
        o_ref[...] = jnp.dot(x_ref[...], w_ref[...],
                             preferred_element_type=jnp.float32)

    return pl.pallas_call(
        body,
        out_shape=jax.ShapeDtypeStruct((x.shape[0], w.shape[1]), jnp.float32),
    )(x, w)


def _mid_tc(p, w1):
    n = p.shape[1]

    def body(p_ref, w1_ref, o_ref):
        h = jnp.maximum(p_ref[0] + p_ref[1], 0.0)
        o_ref[...] = jnp.dot(h, w1_ref[...],
                             preferred_element_type=jnp.float32)

    return pl.pallas_call(
        body,
        out_shape=jax.ShapeDtypeStruct((n, w1.shape[1]), jnp.float32),
    )(p, w1)


def _loss_tc(q, label, mask2d):
    def body(q_ref, y_ref, m_ref, loss_ref, acc_ref):
        logits = q_ref[0] + q_ref[1]
        y = y_ref[...]
        m = m_ref[...]
        k = logits.shape[1]
        rowmax = jnp.max(logits, axis=1, keepdims=True)
        sh = logits - rowmax
        lse = jnp.log(jnp.sum(jnp.exp(sh), axis=1, keepdims=True))
        ce = -jnp.sum(y * (sh - lse), axis=1, keepdims=True)
        ii = lax.broadcasted_iota(jnp.int32, logits.shape, 1)
        am_l = jnp.min(jnp.where(logits >= rowmax, ii, k), axis=1,
                       keepdims=True)
        ymax = jnp.max(y, axis=1, keepdims=True)
        am_y = jnp.min(jnp.where(y >= ymax, ii, k), axis=1, keepdims=True)
        correct = (am_l == am_y).astype(jnp.float32)
        msum = jnp.sum(m)
        loss_ref[...] = (jnp.sum(ce * m) / msum).reshape(1, 1)
        acc_ref[...] = (jnp.sum(correct * m) / msum).reshape(1, 1)

    return pl.pallas_call(
        body,
        out_shape=(jax.ShapeDtypeStruct((1, 1), jnp.float32),
                   jax.ShapeDtypeStruct((1, 1), jnp.float32)),
    )(q, label, mask2d)


def kernel(x, label, mask, edge_index, edge_weight, W0, W1):
    n, d_in = x.shape
    e = edge_index.shape[1]
    src = edge_index[0].astype(jnp.int32)
    dst = edge_index[1].astype(jnp.int32)
    epw = e // NW
    ibc = 5
    nblk = e // (ibc * CH)
    srcr = src.reshape(nblk, ibc, CH)
    dstr = dst.reshape(nblk, ibc, CH)
    cwp = (CH + 15) // 16 * 16
    wr = jnp.pad(edge_weight.reshape(nblk, ibc, CH),
                 ((0, 0), (0, 0), (0, cwp - CH)))

    xw = _matmul_tc(x, W0)
    p = _make_spmm(n, e, W0.shape[1])(xw, srcr, dstr, wr)
    h1 = _mid_tc(p, W1)
    q = _make_spmm(n, e, W1.shape[1])(h1, srcr, dstr, wr)
    loss2, acc2 = _loss_tc(q, label, mask.reshape(n, 1))
    return (loss2[0, 0], acc2[0, 0])
